# baseline (speedup 1.0000x reference)
import functools

import jax
import jax.numpy as jnp
from jax import lax
from jax.experimental import pallas as pl
from jax.experimental.pallas import tpu as pltpu
from jax.experimental.pallas import tpu_sc as plsc

B = 16384
NEG = 20
D = 64
DP = 128
V = 1000000
NC = 2
NS = 16
NW = NC * NS
BPW = B // NW
CHUNK = 16
NCHUNK = BPW // CHUNK
NROWS = CHUNK * NEG
GSZ = 64
NGATH = NROWS // GSZ
TBLK = 32768


def _transpose_body(t_ref, out_ref):
    out_ref[:, 0:D] = jnp.transpose(t_ref[...])


def _pad_rows(table_t):
    grid = (V + TBLK - 1) // TBLK
    return pl.pallas_call(
        _transpose_body,
        grid=(grid,),
        in_specs=[pl.BlockSpec((D, TBLK), lambda i: (0, i))],
        out_specs=pl.BlockSpec((TBLK, DP), lambda i: (i, 0)),
        out_shape=jax.ShapeDtypeStruct((V, DP), jnp.float32),
    )(table_t)


NSLAB = 7812
KFULL = NSLAB // NW
V2 = V // 2


def _sc_transpose_body(t_hbm, tail_hbm, out_hbm,
                       stg_a, stg_b, trs_a, trs_b,
                       sem_ia, sem_ib, sem_oa, sem_ob):
    wid = lax.axis_index("s") * NC + lax.axis_index("c")
    lane = lax.iota(jnp.int32, 16)
    th = [(lane + 16 * j) >> 3 for j in range(4)]
    dl = [(lane + 16 * j) & 7 for j in range(4)]

    def stage(s, stg, sem):
        for t in range(8):
            pltpu.async_copy(
                t_hbm.at[pl.ds(t * 8, 8), pl.ds(s * 128, 128)],
                stg.at[t], sem)

    def wait_in(stg, sem):
        for t in range(8):
            pltpu.make_async_copy(
                t_hbm.at[pl.ds(0, 8), pl.ds(0, 128)], stg.at[t], sem).wait()

    def extract(stg, trs, nlines):
        for ln in range(nlines):
            for half in range(2):
                vv = jnp.full((16,), 2 * ln + half, jnp.int32)
                for j in range(4):
                    trs[ln >> 3, ln & 7, pl.ds(64 * half + 16 * j, 16)] = (
                        plsc.load_gather(stg, [th[j], dl[j], vv]))

    def write(s, trs, sem):
        for g in range(8):
            pltpu.async_copy(
                trs.at[g], out_hbm.at[pl.ds(s * 64 + g * 8, 8), :], sem)

    def wait_out(trs, sem):
        for g in range(8):
            pltpu.make_async_copy(
                trs.at[g], out_hbm.at[pl.ds(0, 8), :], sem).wait()

    def slab(k):
        return wid + NW * k

    stage(slab(0), stg_a, sem_ia)
    stage(slab(1), stg_b, sem_ib)

    def step(t, _):
        k0 = 2 * t
        wait_in(stg_a, sem_ia)

        @pl.when(t > 0)
        def _():
            wait_out(trs_a, sem_oa)

        extract(stg_a, trs_a, 64)
        write(slab(k0), trs_a, sem_oa)
        stage(slab(jnp.minimum(k0 + 2, KFULL - 1)), stg_a, sem_ia)

        wait_in(stg_b, sem_ib)

        @pl.when(t > 0)
        def _():
            wait_out(trs_b, sem_ob)

        extract(stg_b, trs_b, 64)
        write(slab(k0 + 1), trs_b, sem_ob)
        stage(slab(jnp.minimum(k0 + 3, KFULL - 1)), stg_b, sem_ib)

        return 0

    lax.fori_loop(0, KFULL // 2, step, 0)
    wait_in(stg_a, sem_ia)
    wait_in(stg_b, sem_ib)
    wait_out(trs_a, sem_oa)
    wait_out(trs_b, sem_ob)

    @pl.when(wid < NSLAB - NW * KFULL)
    def _():
        s = NW * KFULL + wid
        stage(s, stg_a, sem_ia)
        wait_in(stg_a, sem_ia)
        extract(stg_a, trs_a, 64)
        write(s, trs_a, sem_oa)
        wait_out(trs_a, sem_oa)

    @pl.when(wid == 4)
    def _():
        for t in range(8):
            pltpu.sync_copy(tail_hbm.at[pl.ds(t * 8, 8), :], stg_a.at[t])
        extract(stg_a, trs_a, 32)
        for g in range(4):
            pltpu.sync_copy(
                trs_a.at[g],
                out_hbm.at[pl.ds(NSLAB * 64 + g * 8, 8), :])


@functools.lru_cache(maxsize=1)
def _make_sc_transpose():
    return pl.kernel(
        _sc_transpose_body,
        out_type=jax.ShapeDtypeStruct((V2, DP), jnp.float32),
        mesh=plsc.VectorSubcoreMesh(core_axis_name="c", subcore_axis_name="s",
                                    num_cores=NC, num_subcores=NS),
        compiler_params=pltpu.CompilerParams(
            needs_layout_passes=False, use_tc_tiling_on_sc=True),
        scratch_types=[
            pltpu.VMEM((8, 8, 128), jnp.float32),
            pltpu.VMEM((8, 8, 128), jnp.float32),
            pltpu.VMEM((8, 8, DP), jnp.float32),
            pltpu.VMEM((8, 8, DP), jnp.float32),
            pltpu.SemaphoreType.DMA,
            pltpu.SemaphoreType.DMA,
            pltpu.SemaphoreType.DMA,
            pltpu.SemaphoreType.DMA,
        ],
    )


def _sc_body(center_hbm, context_hbm, negflat_hbm, in_emb_hbm, out_emb_hbm,
             pos_out, neg_out,
             idxc_v, idxx_v, idxn_v, parb_v,
             cen_a, ctx_a, neg_a, cen_b, ctx_b, neg_b,
             pos_v, negs_v, sem_a, sem_b):
    wid = lax.axis_index("s") * NC + lax.axis_index("c")
    base_b = wid * BPW
    base_n = wid * (BPW * NEG)

    pltpu.sync_copy(center_hbm.at[pl.ds(base_b, BPW)], idxc_v)
    pltpu.sync_copy(context_hbm.at[pl.ds(base_b, BPW)], idxx_v)
    pltpu.sync_copy(negflat_hbm.at[pl.ds(base_n, BPW * NEG)], idxn_v)

    for j in range(BPW // 16):
        v = idxc_v[pl.ds(16 * j, 16)]
        parb_v[pl.ds(16 * j, 16)] = (v & 1) * 64
        idxc_v[pl.ds(16 * j, 16)] = v >> 1

    bufs = ((cen_a, ctx_a, neg_a, sem_a), (cen_b, ctx_b, neg_b, sem_b))
    lane = lax.iota(jnp.int32, 16)
    m15 = lane == 15

    def start(c, bufset):
        cen_v, ctx_v, negr_v, sem = bufset
        pltpu.async_copy(
            in_emb_hbm.at[idxc_v.at[pl.ds(c * CHUNK, CHUNK)]], cen_v, sem)
        pltpu.async_copy(
            out_emb_hbm.at[idxx_v.at[pl.ds(c * CHUNK, CHUNK)]], ctx_v, sem)
        for j in range(NGATH):
            pltpu.async_copy(
                out_emb_hbm.at[idxn_v.at[pl.ds(c * NROWS + j * GSZ, GSZ)]],
                negr_v.at[pl.ds(j * GSZ, GSZ)], sem)

    def wait(bufset):
        cen_v, ctx_v, negr_v, sem = bufset
        pltpu.make_async_copy(
            in_emb_hbm.at[idxc_v.at[pl.ds(0, CHUNK)]], cen_v, sem).wait()
        pltpu.make_async_copy(
            out_emb_hbm.at[idxx_v.at[pl.ds(0, CHUNK)]], ctx_v, sem).wait()
        pltpu.make_async_copy(
            out_emb_hbm.at[idxn_v.at[pl.ds(0, NROWS)]], negr_v, sem).wait()

    def compute(c, bufset):
        cen_v, ctx_v, negr_v, _ = bufset

        def _emit_score(ref, pos, vec):
            cs = plsc.cumsum(vec)
            plsc.store_scatter(ref, [jnp.full((16,), pos, jnp.int32)], cs,
                               mask=m15)

        halves = parb_v[pl.ds(c * CHUNK, 16)]

        @plsc.parallel_loop(0, CHUNK)
        def body(b):
            bb = jnp.full((16,), b, jnp.int32)
            base = lax.gather(halves, bb[:, None], lax.GatherDimensionNumbers(offset_dims=(), collapsed_slice_dims=(0,), start_index_map=(0,)), (1,), mode=lax.GatherScatterMode.PROMISE_IN_BOUNDS) + lane
            c0 = plsc.load_gather(cen_v, [bb, base])
            c1 = plsc.load_gather(cen_v, [bb, base + 16])
            c2 = plsc.load_gather(cen_v, [bb, base + 32])
            c3 = plsc.load_gather(cen_v, [bb, base + 48])
            x0 = ctx_v[b, pl.ds(0, 16)]
            x1 = ctx_v[b, pl.ds(16, 16)]
            x2 = ctx_v[b, pl.ds(32, 16)]
            x3 = ctx_v[b, pl.ds(48, 16)]
            p = c0 * x0 + c1 * x1 + c2 * x2 + c3 * x3
            _emit_score(pos_v, c * CHUNK + b, p)
            for n in range(NEG):
                r = b * NEG + n
                n0 = negr_v[r, pl.ds(0, 16)]
                n1 = negr_v[r, pl.ds(16, 16)]
                n2 = negr_v[r, pl.ds(32, 16)]
                n3 = negr_v[r, pl.ds(48, 16)]
                a = n0 * c0 + n1 * c1 + n2 * c2 + n3 * c3
                _emit_score(negs_v, (c * CHUNK + b) * NEG + n, a)

    start(0, bufs[0])
    start(1, bufs[1])

    def step(s, _):
        c0 = 2 * s
        wait(bufs[0])
        compute(c0, bufs[0])

        @pl.when(c0 + 2 < NCHUNK)
        def _():
            start(c0 + 2, bufs[0])

        wait(bufs[1])
        compute(c0 + 1, bufs[1])

        @pl.when(c0 + 3 < NCHUNK)
        def _():
            start(c0 + 3, bufs[1])

        return 0

    lax.fori_loop(0, NCHUNK // 2, step, 0)

    pltpu.sync_copy(pos_v, pos_out.at[pl.ds(base_b, BPW)])
    pltpu.sync_copy(negs_v, neg_out.at[pl.ds(base_n, BPW * NEG)])


@functools.lru_cache(maxsize=1)
def _make_sc_scores():
    return pl.kernel(
        _sc_body,
        out_type=(jax.ShapeDtypeStruct((B,), jnp.float32),
                  jax.ShapeDtypeStruct((B * NEG,), jnp.float32)),
        mesh=plsc.VectorSubcoreMesh(core_axis_name="c", subcore_axis_name="s",
                                    num_cores=NC, num_subcores=NS),
        compiler_params=pltpu.CompilerParams(
            needs_layout_passes=False, use_tc_tiling_on_sc=True),
        scratch_types=[
            pltpu.VMEM((BPW,), jnp.int32),
            pltpu.VMEM((BPW,), jnp.int32),
            pltpu.VMEM((BPW * NEG,), jnp.int32),
            pltpu.VMEM((BPW,), jnp.int32),
            pltpu.VMEM((CHUNK, DP), jnp.float32),
            pltpu.VMEM((CHUNK, DP), jnp.float32),
            pltpu.VMEM((NROWS, DP), jnp.float32),
            pltpu.VMEM((CHUNK, DP), jnp.float32),
            pltpu.VMEM((CHUNK, DP), jnp.float32),
            pltpu.VMEM((NROWS, DP), jnp.float32),
            pltpu.VMEM((BPW,), jnp.float32),
            pltpu.VMEM((BPW * NEG,), jnp.float32),
            pltpu.SemaphoreType.DMA,
            pltpu.SemaphoreType.DMA,
        ],
    )


def _loss_body(pos_ref, neg_ref, out_ref):
    p = pos_ref[...]
    q = neg_ref[...]
    lp = jnp.minimum(p, 0.0) - jnp.log(1.0 + jnp.exp(-jnp.abs(p)))
    ln = jnp.minimum(-q, 0.0) - jnp.log(1.0 + jnp.exp(-jnp.abs(q)))
    out_ref[...] = jnp.reshape(-(jnp.sum(lp) + jnp.sum(ln)) / B, (1, 1))


def kernel(center_words, context_words, negative_words, in_emb, out_emb):
    center = center_words.astype(jnp.int32)
    context = context_words.astype(jnp.int32)
    neg_flat = negative_words.astype(jnp.int32).reshape(B * NEG)
    in_t = in_emb.T
    tail = jnp.pad(in_t[:, NSLAB * 128:], ((0, 0), (0, 128 - (V - NSLAB * 128))))
    in_pad = _make_sc_transpose()(in_t, tail)
    out_pad = _pad_rows(out_emb.T)
    pos_s, neg_s = _make_sc_scores()(center, context, neg_flat,
                                     in_pad, out_pad)
    loss = pl.pallas_call(
        _loss_body,
        out_shape=jax.ShapeDtypeStruct((1, 1), jnp.float32),
    )(pos_s.reshape(B // 128, 128), neg_s.reshape(B * NEG // 128, 128))
    return loss[0, 0]


# speedup vs baseline: 2.9200x; 2.9200x over previous
import functools

import jax
import jax.numpy as jnp
from jax import lax
from jax.experimental import pallas as pl
from jax.experimental.pallas import tpu as pltpu
from jax.experimental.pallas import tpu_sc as plsc

B = 16384
NEG = 20
D = 64
DP = 128
V = 1000000
NC = 2
NS = 16
NW = NC * NS
BPW = B // NW
CHUNK = 16
NCHUNK = BPW // CHUNK
NROWS = CHUNK * NEG
GSZ = 64
NGATH = NROWS // GSZ
TBLK = 32768


def _transpose_body(t_ref, out_ref):
    out_ref[:, 0:D] = jnp.transpose(t_ref[...])


def _pad_rows(table_t):
    grid = (V + TBLK - 1) // TBLK
    return pl.pallas_call(
        _transpose_body,
        grid=(grid,),
        in_specs=[pl.BlockSpec((D, TBLK), lambda i: (0, i))],
        out_specs=pl.BlockSpec((TBLK, DP), lambda i: (i, 0)),
        out_shape=jax.ShapeDtypeStruct((V, DP), jnp.float32),
    )(table_t)


def _sc_body(center_hbm, context_hbm, negflat_hbm, in_emb_hbm, out_emb_hbm,
             pos_out, neg_out,
             idxc_v, idxx_v, idxn_v, cen_a, ctx_a, neg_a, cen_b, ctx_b, neg_b,
             pos_v, negs_v, sem_a, sem_b):
    wid = lax.axis_index("s") * NC + lax.axis_index("c")
    base_b = wid * BPW
    base_n = wid * (BPW * NEG)

    pltpu.sync_copy(center_hbm.at[pl.ds(base_b, BPW)], idxc_v)
    pltpu.sync_copy(context_hbm.at[pl.ds(base_b, BPW)], idxx_v)
    pltpu.sync_copy(negflat_hbm.at[pl.ds(base_n, BPW * NEG)], idxn_v)

    bufs = ((cen_a, ctx_a, neg_a, sem_a), (cen_b, ctx_b, neg_b, sem_b))
    m15 = lax.iota(jnp.int32, 16) == 15

    def start(c, bufset):
        cen_v, ctx_v, negr_v, sem = bufset
        pltpu.async_copy(
            in_emb_hbm.at[idxc_v.at[pl.ds(c * CHUNK, CHUNK)]], cen_v, sem)
        pltpu.async_copy(
            out_emb_hbm.at[idxx_v.at[pl.ds(c * CHUNK, CHUNK)]], ctx_v, sem)
        for j in range(NGATH):
            pltpu.async_copy(
                out_emb_hbm.at[idxn_v.at[pl.ds(c * NROWS + j * GSZ, GSZ)]],
                negr_v.at[pl.ds(j * GSZ, GSZ)], sem)

    def wait(bufset):
        cen_v, ctx_v, negr_v, sem = bufset
        pltpu.make_async_copy(
            in_emb_hbm.at[idxc_v.at[pl.ds(0, CHUNK)]], cen_v, sem).wait()
        pltpu.make_async_copy(
            out_emb_hbm.at[idxx_v.at[pl.ds(0, CHUNK)]], ctx_v, sem).wait()
        pltpu.make_async_copy(
            out_emb_hbm.at[idxn_v.at[pl.ds(0, NROWS)]], negr_v, sem).wait()

    def compute(c, bufset):
        cen_v, ctx_v, negr_v, _ = bufset

        def _emit_score(ref, pos, vec):
            cs = plsc.cumsum(vec)
            plsc.store_scatter(ref, [jnp.full((16,), pos, jnp.int32)], cs,
                               mask=m15)

        @plsc.parallel_loop(0, CHUNK)
        def body(b):
            c0 = cen_v[b, pl.ds(0, 16)]
            c1 = cen_v[b, pl.ds(16, 16)]
            c2 = cen_v[b, pl.ds(32, 16)]
            c3 = cen_v[b, pl.ds(48, 16)]
            x0 = ctx_v[b, pl.ds(0, 16)]
            x1 = ctx_v[b, pl.ds(16, 16)]
            x2 = ctx_v[b, pl.ds(32, 16)]
            x3 = ctx_v[b, pl.ds(48, 16)]
            p = c0 * x0 + c1 * x1 + c2 * x2 + c3 * x3
            _emit_score(pos_v, c * CHUNK + b, p)
            for n in range(NEG):
                r = b * NEG + n
                n0 = negr_v[r, pl.ds(0, 16)]
                n1 = negr_v[r, pl.ds(16, 16)]
                n2 = negr_v[r, pl.ds(32, 16)]
                n3 = negr_v[r, pl.ds(48, 16)]
                a = n0 * c0 + n1 * c1 + n2 * c2 + n3 * c3
                _emit_score(negs_v, (c * CHUNK + b) * NEG + n, a)

    start(0, bufs[0])
    start(1, bufs[1])

    def step(s, _):
        c0 = 2 * s
        wait(bufs[0])
        compute(c0, bufs[0])

        @pl.when(c0 + 2 < NCHUNK)
        def _():
            start(c0 + 2, bufs[0])

        wait(bufs[1])
        compute(c0 + 1, bufs[1])

        @pl.when(c0 + 3 < NCHUNK)
        def _():
            start(c0 + 3, bufs[1])

        return 0

    lax.fori_loop(0, NCHUNK // 2, step, 0)

    pltpu.sync_copy(pos_v, pos_out.at[pl.ds(base_b, BPW)])
    pltpu.sync_copy(negs_v, neg_out.at[pl.ds(base_n, BPW * NEG)])


@functools.lru_cache(maxsize=1)
def _make_sc_scores():
    return pl.kernel(
        _sc_body,
        out_type=(jax.ShapeDtypeStruct((B,), jnp.float32),
                  jax.ShapeDtypeStruct((B * NEG,), jnp.float32)),
        mesh=plsc.VectorSubcoreMesh(core_axis_name="c", subcore_axis_name="s",
                                    num_cores=NC, num_subcores=NS),
        compiler_params=pltpu.CompilerParams(
            needs_layout_passes=False, use_tc_tiling_on_sc=True),
        scratch_types=[
            pltpu.VMEM((BPW,), jnp.int32),
            pltpu.VMEM((BPW,), jnp.int32),
            pltpu.VMEM((BPW * NEG,), jnp.int32),
            pltpu.VMEM((CHUNK, DP), jnp.float32),
            pltpu.VMEM((CHUNK, DP), jnp.float32),
            pltpu.VMEM((NROWS, DP), jnp.float32),
            pltpu.VMEM((CHUNK, DP), jnp.float32),
            pltpu.VMEM((CHUNK, DP), jnp.float32),
            pltpu.VMEM((NROWS, DP), jnp.float32),
            pltpu.VMEM((BPW,), jnp.float32),
            pltpu.VMEM((BPW * NEG,), jnp.float32),
            pltpu.SemaphoreType.DMA,
            pltpu.SemaphoreType.DMA,
        ],
    )


def _loss_body(pos_ref, neg_ref, out_ref):
    p = pos_ref[...]
    q = neg_ref[...]
    lp = jnp.minimum(p, 0.0) - jnp.log(1.0 + jnp.exp(-jnp.abs(p)))
    ln = jnp.minimum(-q, 0.0) - jnp.log(1.0 + jnp.exp(-jnp.abs(q)))
    out_ref[...] = jnp.reshape(-(jnp.sum(lp) + jnp.sum(ln)) / B, (1, 1))


def kernel(center_words, context_words, negative_words, in_emb, out_emb):
    center = center_words.astype(jnp.int32)
    context = context_words.astype(jnp.int32)
    neg_flat = negative_words.astype(jnp.int32).reshape(B * NEG)
    in_pad = _pad_rows(in_emb.T)
    out_pad = _pad_rows(out_emb.T)
    pos_s, neg_s = _make_sc_scores()(center, context, neg_flat,
                                     in_pad, out_pad)
    loss = pl.pallas_call(
        _loss_body,
        out_shape=jax.ShapeDtypeStruct((1, 1), jnp.float32),
    )(pos_s.reshape(B // 128, 128), neg_s.reshape(B * NEG // 128, 128))
    return loss[0, 0]
